# baseline (speedup 1.0000x reference)
import functools

import jax
import jax.numpy as jnp
from jax import lax
from jax.experimental import pallas as pl
from jax.experimental.pallas import tpu as pltpu
from jax.experimental.pallas import tpu_sc as plsc

BATCH = 10000
DIM = 512
NW = 32
GROUPS = BATCH // 8
BASE_G = GROUPS // NW
EXTRA = GROUPS - BASE_G * NW
SPAN_ROWS = (BASE_G + 1) * 8
CHUNK = 64
NCHUNK = SPAN_ROWS // CHUNK
NBUF = 3


def _sc_copy(x_hbm, o_hbm, buf, sem_in, sem_out):
    c = lax.axis_index("c")
    s = lax.axis_index("s")
    wid = s * 2 + c
    start = 8 * (wid * BASE_G + jnp.minimum(wid, EXTRA))
    start = jnp.minimum(start, BATCH - SPAN_ROWS)

    def in_copy(j):
        return pltpu.make_async_copy(
            x_hbm.at[pl.ds(start + j * CHUNK, CHUNK), :],
            buf.at[j % NBUF],
            sem_in.at[j % NBUF],
        )

    def out_copy(j):
        return pltpu.make_async_copy(
            buf.at[j % NBUF],
            o_hbm.at[pl.ds(start + j * CHUNK, CHUNK), :],
            sem_out.at[j % NBUF],
        )

    for j in range(min(NBUF, NCHUNK)):
        in_copy(j).start()
    for j in range(NCHUNK):
        in_copy(j).wait()
        out_copy(j).start()
        if j + NBUF < NCHUNK:
            out_copy(j).wait()
            in_copy(j + NBUF).start()
    for j in range(max(NCHUNK - NBUF, 0), NCHUNK):
        out_copy(j).wait()


def kernel(x, ind, mask, sampled, embed):
    del ind, mask, sampled, embed
    mesh = plsc.VectorSubcoreMesh(core_axis_name="c", subcore_axis_name="s")
    k = functools.partial(
        pl.kernel,
        mesh=mesh,
        out_type=jax.ShapeDtypeStruct((BATCH, DIM), jnp.float32),
        scratch_types=[
            pltpu.VMEM((NBUF, CHUNK, DIM), jnp.float32),
            pltpu.SemaphoreType.DMA((NBUF,)),
            pltpu.SemaphoreType.DMA((NBUF,)),
        ],
    )(_sc_copy)
    return k(x)


# speedup vs baseline: 2.6250x; 2.6250x over previous
import jax
import jax.numpy as jnp
from jax.experimental import pallas as pl

BATCH = 10000
DIM = 512
ROWS_PER_BLOCK = 5000


def _copy_body(x_ref, o_ref):
    o_ref[...] = x_ref[...]


def kernel(x, ind, mask, sampled, embed):
    del ind, mask, sampled, embed
    return pl.pallas_call(
        _copy_body,
        grid=(BATCH // ROWS_PER_BLOCK,),
        in_specs=[pl.BlockSpec((ROWS_PER_BLOCK, DIM), lambda i: (i, 0))],
        out_specs=pl.BlockSpec((ROWS_PER_BLOCK, DIM), lambda i: (i, 0)),
        out_shape=jax.ShapeDtypeStruct((BATCH, DIM), jnp.float32),
    )(x)
